# baseline (speedup 1.0000x reference)
import functools

import jax
import jax.numpy as jnp
from jax import lax
from jax.experimental import pallas as pl
from jax.experimental.pallas import tpu as pltpu
from jax.experimental.pallas import tpu_sc as plsc

N = 10000
E = 320000
D = 128
H = 64
C = 10
G = 128

NC = 2
NS = 16
CHUNK = 128
EPT = E // NS
NBUF = 4
PREF = 2
NCHUNKS = NBUF * (-(-EPT // (CHUNK * NBUF)))
NG = NCHUNKS // NBUF
EPT_PAD = NCHUNKS * CHUNK
RPT = 632
NPAD = NS * RPT
DEG_W = 8
BN = 2000
NB = N // BN

@functools.cache
def _sc_kernels():
    mesh = plsc.VectorSubcoreMesh(
        core_axis_name="c", subcore_axis_name="s",
        num_cores=NC, num_subcores=NS)

    @functools.partial(
        pl.kernel,
        out_type=jax.ShapeDtypeStruct((NC, NPAD, DEG_W), jnp.float32),
        mesh=mesh,
        scratch_types=[
            pltpu.VMEM((NCHUNKS, CHUNK), jnp.int32),
            pltpu.VMEM((CHUNK, DEG_W), jnp.float32),
            pltpu.VMEM_SHARED((NPAD, DEG_W), jnp.float32),
        ],
        compiler_params=pltpu.CompilerParams(use_tc_tiling_on_sc=False),
    )
    def deg_sc(dst_hbm, zeros_hbm, ones_hbm, out_hbm, idx_d, onesb, acc):
        c = lax.axis_index("c")
        s = lax.axis_index("s")
        pltpu.sync_copy(dst_hbm.at[c, s], idx_d)
        pltpu.sync_copy(ones_hbm, onesb)
        pltpu.sync_copy(zeros_hbm.at[pl.ds(s * RPT, RPT)],
                        acc.at[pl.ds(s * RPT, RPT)])
        plsc.subcore_barrier()

        def body(i, carry):
            pltpu.sync_copy(onesb, acc.at[idx_d.at[i]], add=True)
            return carry

        lax.fori_loop(0, NCHUNKS, body, 0)
        plsc.subcore_barrier()
        pltpu.sync_copy(acc.at[pl.ds(s * RPT, RPT)],
                        out_hbm.at[c, pl.ds(s * RPT, RPT)])

    @functools.partial(
        pl.kernel,
        out_type=jax.ShapeDtypeStruct((NC, NPAD, H), jnp.float32),
        mesh=mesh,
        scratch_types=[
            pltpu.VMEM((NCHUNKS, CHUNK), jnp.int32),
            pltpu.VMEM((NCHUNKS, CHUNK), jnp.int32),
            [pltpu.VMEM((CHUNK, H), jnp.float32) for _ in range(NBUF)],
            pltpu.VMEM_SHARED((NPAD, H), jnp.float32),
            pltpu.SemaphoreType.DMA((NBUF,)),
            pltpu.SemaphoreType.DMA((NBUF,)),
        ],
        compiler_params=pltpu.CompilerParams(use_tc_tiling_on_sc=False),
    )
    def agg_sc(u_hbm, src_hbm, dst_hbm, zeros_hbm, out_hbm,
               idx_s, idx_d, rows, acc, gsem, ssem):
        c = lax.axis_index("c")
        s = lax.axis_index("s")
        pltpu.sync_copy(src_hbm.at[c, s], idx_s)
        pltpu.sync_copy(dst_hbm.at[c, s], idx_d)
        pltpu.sync_copy(zeros_hbm.at[pl.ds(s * RPT, RPT)],
                        acc.at[pl.ds(s * RPT, RPT)])
        plsc.subcore_barrier()

        def fire_g(i, j):
            pltpu.async_copy(u_hbm.at[idx_s.at[i]], rows[j], gsem.at[j])

        def wait_g(i, j):
            pltpu.make_async_copy(u_hbm.at[idx_s.at[i]], rows[j],
                                  gsem.at[j]).wait()

        def fire_s(i, j):
            pltpu.async_copy(rows[j], acc.at[idx_d.at[i]], ssem.at[j],
                             add=True)

        def wait_s(i, j):
            pltpu.make_async_copy(rows[j], acc.at[idx_d.at[i]],
                                  ssem.at[j]).wait()

        def step(it, j, first, last):
            i = it * NBUF + j
            k = i + PREF
            jk = (j + PREF) % NBUF
            if not last:
                if not (first and j < PREF):
                    wait_s(k - NBUF, jk)
                fire_g(k, jk)
            elif j < PREF:
                wait_s(k - NBUF, jk)
                fire_g(k, jk)
            wait_g(i, j)
            fire_s(i, j)

        for j in range(PREF):
            fire_g(j, j)
        for j in range(NBUF):
            step(0, j, True, False)

        def group(it, carry):
            for j in range(NBUF):
                step(it, j, False, False)
            return carry

        lax.fori_loop(1, NG - 1, group, 0)
        for j in range(NBUF):
            step(NG - 1, j, False, True)
        for j in range(NBUF):
            wait_s(NCHUNKS - NBUF + j, j)

        plsc.subcore_barrier()
        pltpu.sync_copy(acc.at[pl.ds(s * RPT, RPT)],
                        out_hbm.at[c, pl.ds(s * RPT, RPT)])

    return deg_sc, agg_sc


def _tc_a(xs, Ws, degp):
    def body(x_ref, w_ref, degp_ref, u_ref, dinv_ref):
        deg = jnp.sum(degp_ref[0], axis=1, keepdims=True) + 1.0
        dinv = lax.rsqrt(deg)
        xw = jnp.dot(x_ref[0], w_ref[0], preferred_element_type=jnp.float32)
        u_ref[0] = xw * dinv
        dinv_ref[0] = dinv

    return pl.pallas_call(
        body,
        grid=(2, NB),
        in_specs=[
            pl.BlockSpec((1, BN, D), lambda b, i: (b, i, 0)),
            pl.BlockSpec((1, D, H), lambda b, i: (b, 0, 0)),
            pl.BlockSpec((1, BN, DEG_W), lambda b, i: (b, i, 0)),
        ],
        out_specs=[
            pl.BlockSpec((1, BN, H), lambda b, i: (b, i, 0)),
            pl.BlockSpec((1, BN, 1), lambda b, i: (b, i, 0)),
        ],
        out_shape=[
            jax.ShapeDtypeStruct((2, N, H), jnp.float32),
            jax.ShapeDtypeStruct((2, N, 1), jnp.float32),
        ],
    )(xs, Ws, degp)


def _tc_b(agg, u, dinv, bias, W):
    def body(agg_ref, u_ref, dinv_ref, b_ref, w_ref, un_ref):
        dinv = dinv_ref[0]
        h = jnp.maximum(dinv * (agg_ref[0] + u_ref[0]) + b_ref[0], 0.0)
        un_ref[0] = jnp.dot(h, w_ref[...],
                            preferred_element_type=jnp.float32) * dinv

    return pl.pallas_call(
        body,
        grid=(2, NB),
        in_specs=[
            pl.BlockSpec((1, BN, H), lambda b, i: (b, i, 0)),
            pl.BlockSpec((1, BN, H), lambda b, i: (b, i, 0)),
            pl.BlockSpec((1, BN, 1), lambda b, i: (b, i, 0)),
            pl.BlockSpec((1, 1, H), lambda b, i: (b, 0, 0)),
            pl.BlockSpec((H, H), lambda b, i: (0, 0)),
        ],
        out_specs=pl.BlockSpec((1, BN, H), lambda b, i: (b, i, 0)),
        out_shape=jax.ShapeDtypeStruct((2, N, H), jnp.float32),
    )(agg, u, dinv, bias, W)


def _tc_c(agg, u, dinv, bias, bts):
    def body(agg_ref, u_ref, dinv_ref, b_ref, bt_ref, s_ref, cnt_ref):
        i = pl.program_id(1)
        dinv = dinv_ref[0]
        h = jnp.maximum(dinv * (agg_ref[0] + u_ref[0]) + b_ref[0], 0.0)
        bt = bt_ref[0]
        oh = (bt == lax.broadcasted_iota(jnp.int32, (BN, G), 1)
              ).astype(jnp.float32)
        sp = lax.dot_general(oh, h, (((0,), (0,)), ((), ())),
                             preferred_element_type=jnp.float32)
        cp = lax.dot_general(oh, jnp.ones((BN, 1), jnp.float32),
                             (((0,), (0,)), ((), ())),
                             preferred_element_type=jnp.float32)

        @pl.when(i == 0)
        def _():
            s_ref[0] = sp
            cnt_ref[0] = cp

        @pl.when(i > 0)
        def _():
            s_ref[0] += sp
            cnt_ref[0] += cp

    return pl.pallas_call(
        body,
        grid=(2, NB),
        in_specs=[
            pl.BlockSpec((1, BN, H), lambda b, i: (b, i, 0)),
            pl.BlockSpec((1, BN, H), lambda b, i: (b, i, 0)),
            pl.BlockSpec((1, BN, 1), lambda b, i: (b, i, 0)),
            pl.BlockSpec((1, 1, H), lambda b, i: (b, 0, 0)),
            pl.BlockSpec((1, BN, 1), lambda b, i: (b, i, 0)),
        ],
        out_specs=[
            pl.BlockSpec((1, G, H), lambda b, i: (b, 0, 0)),
            pl.BlockSpec((1, G, 1), lambda b, i: (b, 0, 0)),
        ],
        out_shape=[
            jax.ShapeDtypeStruct((2, G, H), jnp.float32),
            jax.ShapeDtypeStruct((2, G, 1), jnp.float32),
        ],
    )(agg, u, dinv, bias, bts)


def _tc_d(S, cnt, L1s, bl1s, L2s, bl2s):
    def body(s_ref, cnt_ref, l1_ref, bl1_ref, l2_ref, bl2_ref,
             pooled_ref, y_ref):
        cnt = jnp.maximum(cnt_ref[0], 1.0)
        pooled = s_ref[0] / cnt
        t = jnp.maximum(
            jnp.dot(pooled, l1_ref[0], preferred_element_type=jnp.float32)
            + bl1_ref[0], 0.0)
        y_ref[0] = jnp.dot(t, l2_ref[0],
                           preferred_element_type=jnp.float32) + bl2_ref[0]
        pooled_ref[0] = pooled

    return pl.pallas_call(
        body,
        grid=(2,),
        in_specs=[
            pl.BlockSpec((1, G, H), lambda b: (b, 0, 0)),
            pl.BlockSpec((1, G, 1), lambda b: (b, 0, 0)),
            pl.BlockSpec((1, H, H), lambda b: (b, 0, 0)),
            pl.BlockSpec((1, 1, H), lambda b: (b, 0, 0)),
            pl.BlockSpec((1, H, C), lambda b: (b, 0, 0)),
            pl.BlockSpec((1, 1, C), lambda b: (b, 0, 0)),
        ],
        out_specs=[
            pl.BlockSpec((1, G, H), lambda b: (b, 0, 0)),
            pl.BlockSpec((1, G, C), lambda b: (b, 0, 0)),
        ],
        out_shape=[
            jax.ShapeDtypeStruct((2, G, H), jnp.float32),
            jax.ShapeDtypeStruct((2, G, C), jnp.float32),
        ],
    )(S, cnt, L1s, bl1s, L2s, bl2s)


def _prep_edges(ei, soff):
    srcp = jnp.pad(ei[0].reshape(NS, EPT), ((0, 0), (0, EPT_PAD - EPT)),
                   constant_values=0) + soff
    dstp = jnp.pad(ei[1].reshape(NS, EPT), ((0, 0), (0, EPT_PAD - EPT)),
                   constant_values=N)
    return (srcp.reshape(NS, NCHUNKS, CHUNK).astype(jnp.int32),
            dstp.reshape(NS, NCHUNKS, CHUNK).astype(jnp.int32))


def kernel(x, edge_index, batch, x2, edge_index2, batch2,
           W1a, b1a, W1b, b1b, Wc1, bc1, Wc2, bc2,
           L1a, bl1a, L2a, bl2a, L1b, bl1b, L2b, bl2b):
    f32 = jnp.float32
    xs = jnp.stack([x, x2])

    sa, da = _prep_edges(edge_index, 0)
    sb, db = _prep_edges(edge_index2, N)
    srct = jnp.stack([sa, sb])
    dstt = jnp.stack([da, db])

    z_deg = jnp.zeros((NPAD, DEG_W), f32)
    ones_r = jnp.full((CHUNK, DEG_W), 1.0 / DEG_W, f32)
    z_agg = jnp.zeros((NPAD, H), f32)

    deg_sc, agg_sc = _sc_kernels()
    degp = deg_sc(dstt, z_deg, ones_r)
    u, dinv = _tc_a(xs, jnp.stack([W1a, W1b]), degp[:, :N])

    b1 = jnp.stack([b1a, b1b])[:, None, :]
    bc1s = jnp.stack([bc1, bc1])[:, None, :]
    bc2s = jnp.stack([bc2, bc2])[:, None, :]

    agg1 = agg_sc(u.reshape(2 * N, H), srct, dstt, z_agg)
    u2 = _tc_b(agg1[:, :N], u, dinv, b1, Wc1)
    agg2 = agg_sc(u2.reshape(2 * N, H), srct, dstt, z_agg)
    u3 = _tc_b(agg2[:, :N], u2, dinv, bc1s, Wc2)
    agg3 = agg_sc(u3.reshape(2 * N, H), srct, dstt, z_agg)

    S, cnt = _tc_c(agg3[:, :N], u3, dinv, bc2s,
                   jnp.stack([batch, batch2]).astype(jnp.int32)[..., None])
    pooled, y = _tc_d(S, cnt,
                      jnp.stack([L1a, L1b]),
                      jnp.stack([bl1a, bl1b])[:, None, :],
                      jnp.stack([L2a, L2b]),
                      jnp.stack([bl2a, bl2b])[:, None, :])
    return (pooled, y)


# speedup vs baseline: 1.0278x; 1.0278x over previous
import functools

import jax
import jax.numpy as jnp
from jax import lax
from jax.experimental import pallas as pl
from jax.experimental.pallas import tpu as pltpu
from jax.experimental.pallas import tpu_sc as plsc

N = 10000
E = 320000
D = 128
H = 64
C = 10
G = 128

NC = 2
NS = 16
CHUNK = 128
EPT = E // NS
NBUF = 4
PREF = 2
NCHUNKS = NBUF * (-(-EPT // (CHUNK * NBUF)))
NG = NCHUNKS // NBUF
EPT_PAD = NCHUNKS * CHUNK
RPT = 632
NPAD = NS * RPT
DEG_W = 8
BN = 2000
NB = N // BN

@functools.cache
def _sc_kernels():
    mesh = plsc.VectorSubcoreMesh(
        core_axis_name="c", subcore_axis_name="s",
        num_cores=NC, num_subcores=NS)

    @functools.partial(
        pl.kernel,
        out_type=jax.ShapeDtypeStruct((NC, NPAD, DEG_W), jnp.float32),
        mesh=mesh,
        scratch_types=[
            pltpu.VMEM((NCHUNKS, CHUNK), jnp.int32),
            pltpu.VMEM((CHUNK, DEG_W), jnp.float32),
            pltpu.VMEM_SHARED((NPAD, DEG_W), jnp.float32),
        ],
        compiler_params=pltpu.CompilerParams(use_tc_tiling_on_sc=False),
    )
    def deg_sc(dst_hbm, zeros_hbm, ones_hbm, out_hbm, idx_d, onesb, acc):
        c = lax.axis_index("c")
        s = lax.axis_index("s")
        pltpu.sync_copy(dst_hbm.at[c, s], idx_d)
        pltpu.sync_copy(ones_hbm, onesb)
        pltpu.sync_copy(zeros_hbm.at[pl.ds(s * RPT, RPT)],
                        acc.at[pl.ds(s * RPT, RPT)])
        plsc.subcore_barrier()

        def body(i, carry):
            pltpu.sync_copy(onesb, acc.at[idx_d.at[i]], add=True)
            return carry

        lax.fori_loop(0, NCHUNKS, body, 0)
        plsc.subcore_barrier()
        pltpu.sync_copy(acc.at[pl.ds(s * RPT, RPT)],
                        out_hbm.at[c, pl.ds(s * RPT, RPT)])

    @functools.partial(
        pl.kernel,
        out_type=jax.ShapeDtypeStruct((NC, NPAD, H), jnp.float32),
        mesh=mesh,
        scratch_types=[
            pltpu.VMEM((NCHUNKS, CHUNK), jnp.int32),
            pltpu.VMEM((NCHUNKS, CHUNK), jnp.int32),
            [pltpu.VMEM((CHUNK, H), jnp.float32) for _ in range(NBUF)],
            pltpu.VMEM_SHARED((NPAD, H), jnp.float32),
            pltpu.SemaphoreType.DMA((NBUF,)),
            pltpu.SemaphoreType.DMA((NBUF,)),
        ],
        compiler_params=pltpu.CompilerParams(use_tc_tiling_on_sc=False),
    )
    def agg_sc(u_hbm, src_hbm, dst_hbm, zeros_hbm, out_hbm,
               idx_s, idx_d, rows, acc, gsem, ssem):
        c = lax.axis_index("c")
        s = lax.axis_index("s")
        pltpu.sync_copy(src_hbm.at[c, s], idx_s)
        pltpu.sync_copy(dst_hbm.at[c, s], idx_d)
        pltpu.sync_copy(zeros_hbm.at[pl.ds(s * RPT, RPT)],
                        acc.at[pl.ds(s * RPT, RPT)])
        plsc.subcore_barrier()

        def fire_g(i, j):
            pltpu.async_copy(u_hbm.at[idx_s.at[i]], rows[j], gsem.at[j])

        def wait_g(i, j):
            pltpu.make_async_copy(u_hbm.at[idx_s.at[i]], rows[j],
                                  gsem.at[j]).wait()

        def fire_s(i, j):
            pltpu.async_copy(rows[j], acc.at[idx_d.at[i]], ssem.at[j],
                             add=True)

        def wait_s(i, j):
            pltpu.make_async_copy(rows[j], acc.at[idx_d.at[i]],
                                  ssem.at[j]).wait()

        def step(it, j, first, last):
            i = it * NBUF + j
            k = i + PREF
            jk = (j + PREF) % NBUF
            if not last:
                if not (first and j < PREF):
                    wait_s(k - NBUF, jk)
                fire_g(k, jk)
            elif j < PREF:
                wait_s(k - NBUF, jk)
                fire_g(k, jk)
            wait_g(i, j)
            fire_s(i, j)

        for j in range(PREF):
            fire_g(j, j)
        for j in range(NBUF):
            step(0, j, True, False)

        def group(it, carry):
            for j in range(NBUF):
                step(it, j, False, False)
            return carry

        lax.fori_loop(1, NG - 1, group, 0)
        for j in range(NBUF):
            step(NG - 1, j, False, True)
        for j in range(NBUF):
            wait_s(NCHUNKS - NBUF + j, j)

        plsc.subcore_barrier()
        pltpu.sync_copy(acc.at[pl.ds(s * RPT, RPT)],
                        out_hbm.at[c, pl.ds(s * RPT, RPT)])

    return deg_sc, agg_sc


def _tc_a(xs, Ws, degp):
    def body(x_ref, w_ref, degp_ref, u_ref, dinv_ref):
        deg = jnp.sum(degp_ref[0], axis=1, keepdims=True) + 1.0
        dinv = lax.rsqrt(deg)
        xw = jnp.dot(x_ref[0], w_ref[0], preferred_element_type=jnp.float32)
        u_ref[0] = xw * dinv
        dinv_ref[0] = dinv

    return pl.pallas_call(
        body,
        grid=(2, NB),
        in_specs=[
            pl.BlockSpec((1, BN, D), lambda b, i: (b, i, 0)),
            pl.BlockSpec((1, D, H), lambda b, i: (b, 0, 0)),
            pl.BlockSpec((1, BN, DEG_W), lambda b, i: (b, i, 0)),
        ],
        out_specs=[
            pl.BlockSpec((1, BN, H), lambda b, i: (b, i, 0)),
            pl.BlockSpec((1, BN, 1), lambda b, i: (b, i, 0)),
        ],
        out_shape=[
            jax.ShapeDtypeStruct((2, N, H), jnp.float32),
            jax.ShapeDtypeStruct((2, N, 1), jnp.float32),
        ],
    )(xs, Ws, degp)


def _tc_b(agg, u, dinv, bias, W):
    def body(agg_ref, u_ref, dinv_ref, b_ref, w_ref, un_ref):
        dinv = dinv_ref[0]
        h = jnp.maximum(dinv * (agg_ref[0] + u_ref[0]) + b_ref[0], 0.0)
        un_ref[0] = jnp.dot(h, w_ref[...],
                            preferred_element_type=jnp.float32) * dinv

    return pl.pallas_call(
        body,
        grid=(2, NB),
        in_specs=[
            pl.BlockSpec((1, BN, H), lambda b, i: (b, i, 0)),
            pl.BlockSpec((1, BN, H), lambda b, i: (b, i, 0)),
            pl.BlockSpec((1, BN, 1), lambda b, i: (b, i, 0)),
            pl.BlockSpec((1, 1, H), lambda b, i: (b, 0, 0)),
            pl.BlockSpec((H, H), lambda b, i: (0, 0)),
        ],
        out_specs=pl.BlockSpec((1, BN, H), lambda b, i: (b, i, 0)),
        out_shape=jax.ShapeDtypeStruct((2, N, H), jnp.float32),
    )(agg, u, dinv, bias, W)


def _tc_c(agg, u, dinv, bias, bts):
    def body(agg_ref, u_ref, dinv_ref, b_ref, bt_ref, s_ref, cnt_ref):
        i = pl.program_id(1)
        dinv = dinv_ref[0]
        h = jnp.maximum(dinv * (agg_ref[0] + u_ref[0]) + b_ref[0], 0.0)
        bt = bt_ref[0]
        oh = (bt == lax.broadcasted_iota(jnp.int32, (BN, G), 1)
              ).astype(jnp.float32)
        sp = lax.dot_general(oh, h, (((0,), (0,)), ((), ())),
                             preferred_element_type=jnp.float32)
        cp = lax.dot_general(oh, jnp.ones((BN, 1), jnp.float32),
                             (((0,), (0,)), ((), ())),
                             preferred_element_type=jnp.float32)

        @pl.when(i == 0)
        def _():
            s_ref[0] = sp
            cnt_ref[0] = cp

        @pl.when(i > 0)
        def _():
            s_ref[0] += sp
            cnt_ref[0] += cp

    return pl.pallas_call(
        body,
        grid=(2, NB),
        in_specs=[
            pl.BlockSpec((1, BN, H), lambda b, i: (b, i, 0)),
            pl.BlockSpec((1, BN, H), lambda b, i: (b, i, 0)),
            pl.BlockSpec((1, BN, 1), lambda b, i: (b, i, 0)),
            pl.BlockSpec((1, 1, H), lambda b, i: (b, 0, 0)),
            pl.BlockSpec((1, BN, 1), lambda b, i: (b, i, 0)),
        ],
        out_specs=[
            pl.BlockSpec((1, G, H), lambda b, i: (b, 0, 0)),
            pl.BlockSpec((1, G, 1), lambda b, i: (b, 0, 0)),
        ],
        out_shape=[
            jax.ShapeDtypeStruct((2, G, H), jnp.float32),
            jax.ShapeDtypeStruct((2, G, 1), jnp.float32),
        ],
    )(agg, u, dinv, bias, bts)


def _tc_d(S, cnt, L1s, bl1s, L2s, bl2s):
    def body(s_ref, cnt_ref, l1_ref, bl1_ref, l2_ref, bl2_ref,
             pooled_ref, y_ref):
        cnt = jnp.maximum(cnt_ref[0], 1.0)
        pooled = s_ref[0] / cnt
        t = jnp.maximum(
            jnp.dot(pooled, l1_ref[0], preferred_element_type=jnp.float32)
            + bl1_ref[0], 0.0)
        y_ref[0] = jnp.dot(t, l2_ref[0],
                           preferred_element_type=jnp.float32) + bl2_ref[0]
        pooled_ref[0] = pooled

    return pl.pallas_call(
        body,
        grid=(2,),
        in_specs=[
            pl.BlockSpec((1, G, H), lambda b: (b, 0, 0)),
            pl.BlockSpec((1, G, 1), lambda b: (b, 0, 0)),
            pl.BlockSpec((1, H, H), lambda b: (b, 0, 0)),
            pl.BlockSpec((1, 1, H), lambda b: (b, 0, 0)),
            pl.BlockSpec((1, H, C), lambda b: (b, 0, 0)),
            pl.BlockSpec((1, 1, C), lambda b: (b, 0, 0)),
        ],
        out_specs=[
            pl.BlockSpec((1, G, H), lambda b: (b, 0, 0)),
            pl.BlockSpec((1, G, C), lambda b: (b, 0, 0)),
        ],
        out_shape=[
            jax.ShapeDtypeStruct((2, G, H), jnp.float32),
            jax.ShapeDtypeStruct((2, G, C), jnp.float32),
        ],
    )(S, cnt, L1s, bl1s, L2s, bl2s)


def _prep_edges(ei, soff):
    srcp = jnp.pad(ei[0].reshape(NS, EPT), ((0, 0), (0, EPT_PAD - EPT)),
                   constant_values=0) + soff
    dstp = jnp.pad(ei[1].reshape(NS, EPT), ((0, 0), (0, EPT_PAD - EPT)),
                   constant_values=N)
    return (srcp.reshape(NS, NCHUNKS, CHUNK).astype(jnp.int32),
            dstp.reshape(NS, NCHUNKS, CHUNK).astype(jnp.int32))


def kernel(x, edge_index, batch, x2, edge_index2, batch2,
           W1a, b1a, W1b, b1b, Wc1, bc1, Wc2, bc2,
           L1a, bl1a, L2a, bl2a, L1b, bl1b, L2b, bl2b):
    f32 = jnp.float32
    xs = jnp.stack([x, x2])

    sa, da = _prep_edges(edge_index, 0)
    sb, db = _prep_edges(edge_index2, N)
    srct = jnp.stack([sa, sb])
    dstt = jnp.stack([da, db])

    z_deg = jnp.zeros((NPAD, DEG_W), f32)
    ones_r = jnp.full((CHUNK, DEG_W), 1.0 / DEG_W, f32)
    z_agg = jnp.zeros((NPAD, H), f32)

    deg_sc, agg_sc = _sc_kernels()
    degp = deg_sc(dstt, z_deg, ones_r)
    u, dinv = _tc_a(xs, jnp.stack([W1a, W1b]), degp)

    b1 = jnp.stack([b1a, b1b])[:, None, :]
    bc1s = jnp.stack([bc1, bc1])[:, None, :]
    bc2s = jnp.stack([bc2, bc2])[:, None, :]

    agg1 = agg_sc(u.reshape(2 * N, H), srct, dstt, z_agg)
    u2 = _tc_b(agg1, u, dinv, b1, Wc1)
    agg2 = agg_sc(u2.reshape(2 * N, H), srct, dstt, z_agg)
    u3 = _tc_b(agg2, u2, dinv, bc1s, Wc2)
    agg3 = agg_sc(u3.reshape(2 * N, H), srct, dstt, z_agg)

    S, cnt = _tc_c(agg3, u3, dinv, bc2s,
                   jnp.stack([batch, batch2]).astype(jnp.int32)[..., None])
    pooled, y = _tc_d(S, cnt,
                      jnp.stack([L1a, L1b]),
                      jnp.stack([bl1a, bl1b])[:, None, :],
                      jnp.stack([L2a, L2b]),
                      jnp.stack([bl2a, bl2b])[:, None, :])
    return (pooled, y)


# speedup vs baseline: 1.5838x; 1.5410x over previous
import functools

import jax
import jax.numpy as jnp
from jax import lax
from jax.experimental import pallas as pl
from jax.experimental.pallas import tpu as pltpu
from jax.experimental.pallas import tpu_sc as plsc

N = 10000
E = 320000
D = 128
H = 64
C = 10
G = 128

NC = 2
NS = 16
CHUNK = 128
EPT = E // NS
NBUF = 8
PREF = 4
NCHUNKS = NBUF * (-(-EPT // (CHUNK * NBUF)))
NG = NCHUNKS // NBUF
EPT_PAD = NCHUNKS * CHUNK
RPT = 632
NPAD = NS * RPT
DEG_W = 8
BN = 2000
NB = N // BN

@functools.cache
def _sc_kernels():
    mesh = plsc.VectorSubcoreMesh(
        core_axis_name="c", subcore_axis_name="s",
        num_cores=NC, num_subcores=NS)

    @functools.partial(
        pl.kernel,
        out_type=jax.ShapeDtypeStruct((NC, NPAD, DEG_W), jnp.float32),
        mesh=mesh,
        scratch_types=[
            pltpu.VMEM((NCHUNKS, CHUNK), jnp.int32),
            pltpu.VMEM((CHUNK, DEG_W), jnp.float32),
            pltpu.VMEM_SHARED((NPAD, DEG_W), jnp.float32),
        ],
        compiler_params=pltpu.CompilerParams(use_tc_tiling_on_sc=False),
    )
    def deg_sc(dst_hbm, zeros_hbm, ones_hbm, out_hbm, idx_d, onesb, acc):
        c = lax.axis_index("c")
        s = lax.axis_index("s")
        pltpu.sync_copy(dst_hbm.at[c, s], idx_d)
        pltpu.sync_copy(ones_hbm, onesb)
        pltpu.sync_copy(zeros_hbm.at[pl.ds(s * RPT, RPT)],
                        acc.at[pl.ds(s * RPT, RPT)])
        plsc.subcore_barrier()

        def body(i, carry):
            pltpu.sync_copy(onesb, acc.at[idx_d.at[i]], add=True)
            return carry

        lax.fori_loop(0, NCHUNKS, body, 0)
        plsc.subcore_barrier()
        pltpu.sync_copy(acc.at[pl.ds(s * RPT, RPT)],
                        out_hbm.at[c, pl.ds(s * RPT, RPT)])

    @functools.partial(
        pl.kernel,
        out_type=jax.ShapeDtypeStruct((NC, NPAD, H), jnp.bfloat16),
        mesh=mesh,
        scratch_types=[
            pltpu.VMEM((NCHUNKS, CHUNK), jnp.int32),
            pltpu.VMEM((NCHUNKS, CHUNK), jnp.int32),
            [pltpu.VMEM((CHUNK, H), jnp.bfloat16) for _ in range(NBUF)],
            pltpu.VMEM_SHARED((NPAD, H), jnp.bfloat16),
            pltpu.SemaphoreType.DMA((NBUF,)),
            pltpu.SemaphoreType.DMA((NBUF,)),
        ],
        compiler_params=pltpu.CompilerParams(use_tc_tiling_on_sc=False),
    )
    def agg_sc(u_hbm, src_hbm, dst_hbm, zeros_hbm, out_hbm,
               idx_s, idx_d, rows, acc, gsem, ssem):
        c = lax.axis_index("c")
        s = lax.axis_index("s")
        pltpu.sync_copy(src_hbm.at[c, s], idx_s)
        pltpu.sync_copy(dst_hbm.at[c, s], idx_d)
        pltpu.sync_copy(zeros_hbm.at[pl.ds(s * RPT, RPT)],
                        acc.at[pl.ds(s * RPT, RPT)])
        plsc.subcore_barrier()

        def fire_g(i, j):
            pltpu.async_copy(u_hbm.at[idx_s.at[i]], rows[j], gsem.at[j])

        def wait_g(i, j):
            pltpu.make_async_copy(u_hbm.at[idx_s.at[i]], rows[j],
                                  gsem.at[j]).wait()

        def fire_s(i, j):
            pltpu.async_copy(rows[j], acc.at[idx_d.at[i]], ssem.at[j],
                             add=True)

        def wait_s(i, j):
            pltpu.make_async_copy(rows[j], acc.at[idx_d.at[i]],
                                  ssem.at[j]).wait()

        def step(it, j, first, last):
            i = it * NBUF + j
            k = i + PREF
            jk = (j + PREF) % NBUF
            if not last:
                if not (first and j < PREF):
                    wait_s(k - NBUF, jk)
                fire_g(k, jk)
            elif j < PREF:
                wait_s(k - NBUF, jk)
                fire_g(k, jk)
            wait_g(i, j)
            fire_s(i, j)

        for j in range(PREF):
            fire_g(j, j)
        for j in range(NBUF):
            step(0, j, True, False)

        def group(it, carry):
            for j in range(NBUF):
                step(it, j, False, False)
            return carry

        lax.fori_loop(1, NG - 1, group, 0)
        for j in range(NBUF):
            step(NG - 1, j, False, True)
        for j in range(NBUF):
            wait_s(NCHUNKS - NBUF + j, j)

        plsc.subcore_barrier()
        pltpu.sync_copy(acc.at[pl.ds(s * RPT, RPT)],
                        out_hbm.at[c, pl.ds(s * RPT, RPT)])

    return deg_sc, agg_sc


def _tc_a(xs, Ws, degp):
    def body(x_ref, w_ref, degp_ref, u_ref, ub_ref, dinv_ref):
        deg = jnp.sum(degp_ref[0], axis=1, keepdims=True) + 1.0
        dinv = lax.rsqrt(deg)
        xw = jnp.dot(x_ref[0], w_ref[0], preferred_element_type=jnp.float32)
        u = xw * dinv
        u_ref[0] = u
        ub_ref[0] = u.astype(jnp.bfloat16)
        dinv_ref[0] = dinv

    return pl.pallas_call(
        body,
        grid=(2, NB),
        in_specs=[
            pl.BlockSpec((1, BN, D), lambda b, i: (b, i, 0)),
            pl.BlockSpec((1, D, H), lambda b, i: (b, 0, 0)),
            pl.BlockSpec((1, BN, DEG_W), lambda b, i: (b, i, 0)),
        ],
        out_specs=[
            pl.BlockSpec((1, BN, H), lambda b, i: (b, i, 0)),
            pl.BlockSpec((1, BN, H), lambda b, i: (b, i, 0)),
            pl.BlockSpec((1, BN, 1), lambda b, i: (b, i, 0)),
        ],
        out_shape=[
            jax.ShapeDtypeStruct((2, N, H), jnp.float32),
            jax.ShapeDtypeStruct((2, N, H), jnp.bfloat16),
            jax.ShapeDtypeStruct((2, N, 1), jnp.float32),
        ],
    )(xs, Ws, degp)


def _tc_b(agg, u, dinv, bias, W):
    def body(agg_ref, u_ref, dinv_ref, b_ref, w_ref, un_ref, unb_ref):
        dinv = dinv_ref[0]
        agg = agg_ref[0].astype(jnp.float32)
        h = jnp.maximum(dinv * (agg + u_ref[0]) + b_ref[0], 0.0)
        un = jnp.dot(h, w_ref[...],
                     preferred_element_type=jnp.float32) * dinv
        un_ref[0] = un
        unb_ref[0] = un.astype(jnp.bfloat16)

    return pl.pallas_call(
        body,
        grid=(2, NB),
        in_specs=[
            pl.BlockSpec((1, BN, H), lambda b, i: (b, i, 0)),
            pl.BlockSpec((1, BN, H), lambda b, i: (b, i, 0)),
            pl.BlockSpec((1, BN, 1), lambda b, i: (b, i, 0)),
            pl.BlockSpec((1, 1, H), lambda b, i: (b, 0, 0)),
            pl.BlockSpec((H, H), lambda b, i: (0, 0)),
        ],
        out_specs=[
            pl.BlockSpec((1, BN, H), lambda b, i: (b, i, 0)),
            pl.BlockSpec((1, BN, H), lambda b, i: (b, i, 0)),
        ],
        out_shape=[
            jax.ShapeDtypeStruct((2, N, H), jnp.float32),
            jax.ShapeDtypeStruct((2, N, H), jnp.bfloat16),
        ],
    )(agg, u, dinv, bias, W)


def _tc_c(agg, u, dinv, bias, bts):
    def body(agg_ref, u_ref, dinv_ref, b_ref, bt_ref, s_ref, cnt_ref):
        i = pl.program_id(1)
        dinv = dinv_ref[0]
        agg = agg_ref[0].astype(jnp.float32)
        h = jnp.maximum(dinv * (agg + u_ref[0]) + b_ref[0], 0.0)
        bt = bt_ref[0]
        oh = (bt == lax.broadcasted_iota(jnp.int32, (BN, G), 1)
              ).astype(jnp.float32)
        sp = lax.dot_general(oh, h, (((0,), (0,)), ((), ())),
                             preferred_element_type=jnp.float32)
        cp = lax.dot_general(oh, jnp.ones((BN, 1), jnp.float32),
                             (((0,), (0,)), ((), ())),
                             preferred_element_type=jnp.float32)

        @pl.when(i == 0)
        def _():
            s_ref[0] = sp
            cnt_ref[0] = cp

        @pl.when(i > 0)
        def _():
            s_ref[0] += sp
            cnt_ref[0] += cp

    return pl.pallas_call(
        body,
        grid=(2, NB),
        in_specs=[
            pl.BlockSpec((1, BN, H), lambda b, i: (b, i, 0)),
            pl.BlockSpec((1, BN, H), lambda b, i: (b, i, 0)),
            pl.BlockSpec((1, BN, 1), lambda b, i: (b, i, 0)),
            pl.BlockSpec((1, 1, H), lambda b, i: (b, 0, 0)),
            pl.BlockSpec((1, BN, 1), lambda b, i: (b, i, 0)),
        ],
        out_specs=[
            pl.BlockSpec((1, G, H), lambda b, i: (b, 0, 0)),
            pl.BlockSpec((1, G, 1), lambda b, i: (b, 0, 0)),
        ],
        out_shape=[
            jax.ShapeDtypeStruct((2, G, H), jnp.float32),
            jax.ShapeDtypeStruct((2, G, 1), jnp.float32),
        ],
    )(agg, u, dinv, bias, bts)


def _tc_d(S, cnt, L1s, bl1s, L2s, bl2s):
    def body(s_ref, cnt_ref, l1_ref, bl1_ref, l2_ref, bl2_ref,
             pooled_ref, y_ref):
        cnt = jnp.maximum(cnt_ref[0], 1.0)
        pooled = s_ref[0] / cnt
        t = jnp.maximum(
            jnp.dot(pooled, l1_ref[0], preferred_element_type=jnp.float32)
            + bl1_ref[0], 0.0)
        y_ref[0] = jnp.dot(t, l2_ref[0],
                           preferred_element_type=jnp.float32) + bl2_ref[0]
        pooled_ref[0] = pooled

    return pl.pallas_call(
        body,
        grid=(2,),
        in_specs=[
            pl.BlockSpec((1, G, H), lambda b: (b, 0, 0)),
            pl.BlockSpec((1, G, 1), lambda b: (b, 0, 0)),
            pl.BlockSpec((1, H, H), lambda b: (b, 0, 0)),
            pl.BlockSpec((1, 1, H), lambda b: (b, 0, 0)),
            pl.BlockSpec((1, H, C), lambda b: (b, 0, 0)),
            pl.BlockSpec((1, 1, C), lambda b: (b, 0, 0)),
        ],
        out_specs=[
            pl.BlockSpec((1, G, H), lambda b: (b, 0, 0)),
            pl.BlockSpec((1, G, C), lambda b: (b, 0, 0)),
        ],
        out_shape=[
            jax.ShapeDtypeStruct((2, G, H), jnp.float32),
            jax.ShapeDtypeStruct((2, G, C), jnp.float32),
        ],
    )(S, cnt, L1s, bl1s, L2s, bl2s)


def _prep_edges(ei, soff):
    srcp = jnp.pad(ei[0].reshape(NS, EPT), ((0, 0), (0, EPT_PAD - EPT)),
                   constant_values=0) + soff
    dstp = jnp.pad(ei[1].reshape(NS, EPT), ((0, 0), (0, EPT_PAD - EPT)),
                   constant_values=N)
    return (srcp.reshape(NS, NCHUNKS, CHUNK).astype(jnp.int32),
            dstp.reshape(NS, NCHUNKS, CHUNK).astype(jnp.int32))


def kernel(x, edge_index, batch, x2, edge_index2, batch2,
           W1a, b1a, W1b, b1b, Wc1, bc1, Wc2, bc2,
           L1a, bl1a, L2a, bl2a, L1b, bl1b, L2b, bl2b):
    f32 = jnp.float32
    xs = jnp.stack([x, x2])

    sa, da = _prep_edges(edge_index, 0)
    sb, db = _prep_edges(edge_index2, N)
    srct = jnp.stack([sa, sb])
    dstt = jnp.stack([da, db])

    z_deg = jnp.zeros((NPAD, DEG_W), f32)
    ones_r = jnp.full((CHUNK, DEG_W), 1.0 / DEG_W, f32)
    z_agg = jnp.zeros((NPAD, H), jnp.bfloat16)

    deg_sc, agg_sc = _sc_kernels()
    degp = deg_sc(dstt, z_deg, ones_r)
    u, ub, dinv = _tc_a(xs, jnp.stack([W1a, W1b]), degp)

    b1 = jnp.stack([b1a, b1b])[:, None, :]
    bc1s = jnp.stack([bc1, bc1])[:, None, :]
    bc2s = jnp.stack([bc2, bc2])[:, None, :]

    agg1 = agg_sc(ub.reshape(2 * N, H), srct, dstt, z_agg)
    u2, ub2 = _tc_b(agg1, u, dinv, b1, Wc1)
    agg2 = agg_sc(ub2.reshape(2 * N, H), srct, dstt, z_agg)
    u3, ub3 = _tc_b(agg2, u2, dinv, bc1s, Wc2)
    agg3 = agg_sc(ub3.reshape(2 * N, H), srct, dstt, z_agg)

    S, cnt = _tc_c(agg3, u3, dinv, bc2s,
                   jnp.stack([batch, batch2]).astype(jnp.int32)[..., None])
    pooled, y = _tc_d(S, cnt,
                      jnp.stack([L1a, L1b]),
                      jnp.stack([bl1a, bl1b])[:, None, :],
                      jnp.stack([L2a, L2b]),
                      jnp.stack([bl2a, bl2b])[:, None, :])
    return (pooled, y)


# speedup vs baseline: 1.6385x; 1.0346x over previous
import functools

import jax
import jax.numpy as jnp
from jax import lax
from jax.experimental import pallas as pl
from jax.experimental.pallas import tpu as pltpu
from jax.experimental.pallas import tpu_sc as plsc

N = 10000
E = 320000
D = 128
H = 64
C = 10
G = 128

NC = 2
NS = 16
CHUNK = 128
EPT = E // NS
NBUF = 8
PREF = 4
NCHUNKS = NBUF * (-(-EPT // (CHUNK * NBUF)))
NG = NCHUNKS // NBUF
EPT_PAD = NCHUNKS * CHUNK
RPT = 632
NPAD = NS * RPT
DEG_W = 8
BN = 2000
NB = N // BN

@functools.cache
def _sc_kernels():
    mesh = plsc.VectorSubcoreMesh(
        core_axis_name="c", subcore_axis_name="s",
        num_cores=NC, num_subcores=NS)

    @functools.partial(
        pl.kernel,
        out_type=jax.ShapeDtypeStruct((NC, NPAD, DEG_W), jnp.float32),
        mesh=mesh,
        scratch_types=[
            pltpu.VMEM((NCHUNKS, CHUNK), jnp.int32),
            pltpu.VMEM((CHUNK, DEG_W), jnp.float32),
            pltpu.VMEM_SHARED((NPAD, DEG_W), jnp.float32),
            pltpu.SemaphoreType.DMA,
        ],
        compiler_params=pltpu.CompilerParams(use_tc_tiling_on_sc=False),
    )
    def deg_sc(dst_hbm, zeros_hbm, ones_hbm, out_hbm, idx_d, onesb, acc, sem):
        c = lax.axis_index("c")
        s = lax.axis_index("s")
        pltpu.sync_copy(dst_hbm.at[c, s], idx_d)
        pltpu.sync_copy(ones_hbm, onesb)
        pltpu.sync_copy(zeros_hbm.at[pl.ds(s * RPT, RPT)],
                        acc.at[pl.ds(s * RPT, RPT)])
        plsc.subcore_barrier()

        def fire(i):
            pltpu.async_copy(onesb, acc.at[idx_d.at[i]], sem, add=True)

        def wait_one():
            pltpu.make_async_copy(onesb, acc.at[idx_d.at[0]], sem).wait()

        for i in range(NBUF):
            fire(i)

        def body(i, carry):
            wait_one()
            fire(i)
            return carry

        lax.fori_loop(NBUF, NCHUNKS, body, 0)
        for _ in range(NBUF):
            wait_one()
        plsc.subcore_barrier()
        pltpu.sync_copy(acc.at[pl.ds(s * RPT, RPT)],
                        out_hbm.at[c, pl.ds(s * RPT, RPT)])

    @functools.partial(
        pl.kernel,
        out_type=jax.ShapeDtypeStruct((NC, NPAD, H), jnp.bfloat16),
        mesh=mesh,
        scratch_types=[
            pltpu.VMEM((NCHUNKS, CHUNK), jnp.int32),
            pltpu.VMEM((NCHUNKS, CHUNK), jnp.int32),
            [pltpu.VMEM((CHUNK, H), jnp.bfloat16) for _ in range(NBUF)],
            pltpu.VMEM_SHARED((NPAD, H), jnp.bfloat16),
            pltpu.SemaphoreType.DMA((NBUF,)),
            pltpu.SemaphoreType.DMA((NBUF,)),
        ],
        compiler_params=pltpu.CompilerParams(use_tc_tiling_on_sc=False),
    )
    def agg_sc(u_hbm, src_hbm, dst_hbm, zeros_hbm, out_hbm,
               idx_s, idx_d, rows, acc, gsem, ssem):
        c = lax.axis_index("c")
        s = lax.axis_index("s")
        pltpu.sync_copy(src_hbm.at[c, s], idx_s)
        pltpu.sync_copy(dst_hbm.at[c, s], idx_d)
        pltpu.sync_copy(zeros_hbm.at[pl.ds(s * RPT, RPT)],
                        acc.at[pl.ds(s * RPT, RPT)])
        plsc.subcore_barrier()

        def fire_g(i, j):
            pltpu.async_copy(u_hbm.at[idx_s.at[i]], rows[j], gsem.at[j])

        def wait_g(i, j):
            pltpu.make_async_copy(u_hbm.at[idx_s.at[i]], rows[j],
                                  gsem.at[j]).wait()

        def fire_s(i, j):
            pltpu.async_copy(rows[j], acc.at[idx_d.at[i]], ssem.at[j],
                             add=True)

        def wait_s(i, j):
            pltpu.make_async_copy(rows[j], acc.at[idx_d.at[i]],
                                  ssem.at[j]).wait()

        def step(it, j, first, last):
            i = it * NBUF + j
            k = i + PREF
            jk = (j + PREF) % NBUF
            if not last:
                if not (first and j < PREF):
                    wait_s(k - NBUF, jk)
                fire_g(k, jk)
            elif j < PREF:
                wait_s(k - NBUF, jk)
                fire_g(k, jk)
            wait_g(i, j)
            fire_s(i, j)

        for j in range(PREF):
            fire_g(j, j)
        for j in range(NBUF):
            step(0, j, True, False)

        def group(it, carry):
            for j in range(NBUF):
                step(it, j, False, False)
            return carry

        lax.fori_loop(1, NG - 1, group, 0)
        for j in range(NBUF):
            step(NG - 1, j, False, True)
        for j in range(NBUF):
            wait_s(NCHUNKS - NBUF + j, j)

        plsc.subcore_barrier()
        pltpu.sync_copy(acc.at[pl.ds(s * RPT, RPT)],
                        out_hbm.at[c, pl.ds(s * RPT, RPT)])

    return deg_sc, agg_sc


def _tc_a(xs, Ws, degp):
    def body(x_ref, w_ref, degp_ref, ub_ref, dinv_ref):
        deg = jnp.sum(degp_ref[0], axis=1, keepdims=True) + 1.0
        dinv = lax.rsqrt(deg)
        xw = jnp.dot(x_ref[0], w_ref[0], preferred_element_type=jnp.float32)
        ub_ref[...] = (xw * dinv).astype(jnp.bfloat16)
        dinv_ref[0] = dinv

    return pl.pallas_call(
        body,
        grid=(2, NB),
        in_specs=[
            pl.BlockSpec((1, BN, D), lambda b, i: (b, i, 0)),
            pl.BlockSpec((1, D, H), lambda b, i: (b, 0, 0)),
            pl.BlockSpec((1, BN, DEG_W), lambda b, i: (b, i, 0)),
        ],
        out_specs=[
            pl.BlockSpec((BN, H), lambda b, i: (b * NB + i, 0)),
            pl.BlockSpec((1, BN, 1), lambda b, i: (b, i, 0)),
        ],
        out_shape=[
            jax.ShapeDtypeStruct((2 * N, H), jnp.bfloat16),
            jax.ShapeDtypeStruct((2, N, 1), jnp.float32),
        ],
    )(xs, Ws, degp)


def _tc_b(agg, ub, dinv, bias, W):
    def body(agg_ref, ub_ref, dinv_ref, b_ref, w_ref, un_ref):
        dinv = dinv_ref[0]
        pre = (agg_ref[0] + ub_ref[...]).astype(jnp.float32)
        h = jnp.maximum(dinv * pre + b_ref[0], 0.0)
        un = jnp.dot(h, w_ref[...],
                     preferred_element_type=jnp.float32) * dinv
        un_ref[...] = un.astype(jnp.bfloat16)

    return pl.pallas_call(
        body,
        grid=(2, NB),
        in_specs=[
            pl.BlockSpec((1, BN, H), lambda b, i: (b, i, 0)),
            pl.BlockSpec((BN, H), lambda b, i: (b * NB + i, 0)),
            pl.BlockSpec((1, BN, 1), lambda b, i: (b, i, 0)),
            pl.BlockSpec((1, 1, H), lambda b, i: (b, 0, 0)),
            pl.BlockSpec((H, H), lambda b, i: (0, 0)),
        ],
        out_specs=pl.BlockSpec((BN, H), lambda b, i: (b * NB + i, 0)),
        out_shape=jax.ShapeDtypeStruct((2 * N, H), jnp.bfloat16),
    )(agg, ub, dinv, bias, W)


def _tc_c(agg, ub, dinv, bias, bts, L1s, bl1s, L2s, bl2s):
    def body(agg_ref, ub_ref, dinv_ref, b_ref, bt_ref,
             l1_ref, bl1_ref, l2_ref, bl2_ref,
             pooled_ref, y_ref, s_acc, cnt_acc):
        i = pl.program_id(1)
        dinv = dinv_ref[0]
        pre = (agg_ref[0] + ub_ref[...]).astype(jnp.float32)
        h = jnp.maximum(dinv * pre + b_ref[0], 0.0)
        bt = bt_ref[0]
        oh = (bt == lax.broadcasted_iota(jnp.int32, (BN, G), 1)
              ).astype(jnp.float32)
        sp = lax.dot_general(oh, h, (((0,), (0,)), ((), ())),
                             preferred_element_type=jnp.float32)
        cp = lax.dot_general(oh, jnp.ones((BN, 1), jnp.float32),
                             (((0,), (0,)), ((), ())),
                             preferred_element_type=jnp.float32)

        @pl.when(i == 0)
        def _():
            s_acc[...] = sp
            cnt_acc[...] = cp

        @pl.when(i > 0)
        def _():
            s_acc[...] += sp
            cnt_acc[...] += cp

        @pl.when(i == NB - 1)
        def _():
            pooled = s_acc[...] / jnp.maximum(cnt_acc[...], 1.0)
            t = jnp.maximum(
                jnp.dot(pooled, l1_ref[0],
                        preferred_element_type=jnp.float32) + bl1_ref[0], 0.0)
            y_ref[0] = jnp.dot(t, l2_ref[0],
                               preferred_element_type=jnp.float32) + bl2_ref[0]
            pooled_ref[0] = pooled

    return pl.pallas_call(
        body,
        grid=(2, NB),
        in_specs=[
            pl.BlockSpec((1, BN, H), lambda b, i: (b, i, 0)),
            pl.BlockSpec((BN, H), lambda b, i: (b * NB + i, 0)),
            pl.BlockSpec((1, BN, 1), lambda b, i: (b, i, 0)),
            pl.BlockSpec((1, 1, H), lambda b, i: (b, 0, 0)),
            pl.BlockSpec((1, BN, 1), lambda b, i: (b, i, 0)),
            pl.BlockSpec((1, H, H), lambda b, i: (b, 0, 0)),
            pl.BlockSpec((1, 1, H), lambda b, i: (b, 0, 0)),
            pl.BlockSpec((1, H, C), lambda b, i: (b, 0, 0)),
            pl.BlockSpec((1, 1, C), lambda b, i: (b, 0, 0)),
        ],
        out_specs=[
            pl.BlockSpec((1, G, H), lambda b, i: (b, 0, 0)),
            pl.BlockSpec((1, G, C), lambda b, i: (b, 0, 0)),
        ],
        out_shape=[
            jax.ShapeDtypeStruct((2, G, H), jnp.float32),
            jax.ShapeDtypeStruct((2, G, C), jnp.float32),
        ],
        scratch_shapes=[
            pltpu.VMEM((G, H), jnp.float32),
            pltpu.VMEM((G, 1), jnp.float32),
        ],
    )(agg, ub, dinv, bias, bts, L1s, bl1s, L2s, bl2s)


def _prep_edges(ei, soff):
    srcp = jnp.pad(ei[0].reshape(NS, EPT), ((0, 0), (0, EPT_PAD - EPT)),
                   constant_values=0) + soff
    dstp = jnp.pad(ei[1].reshape(NS, EPT), ((0, 0), (0, EPT_PAD - EPT)),
                   constant_values=N)
    return (srcp.reshape(NS, NCHUNKS, CHUNK).astype(jnp.int32),
            dstp.reshape(NS, NCHUNKS, CHUNK).astype(jnp.int32))


def kernel(x, edge_index, batch, x2, edge_index2, batch2,
           W1a, b1a, W1b, b1b, Wc1, bc1, Wc2, bc2,
           L1a, bl1a, L2a, bl2a, L1b, bl1b, L2b, bl2b):
    f32 = jnp.float32
    xs = jnp.stack([x, x2])

    sa, da = _prep_edges(edge_index, 0)
    sb, db = _prep_edges(edge_index2, N)
    srct = jnp.stack([sa, sb])
    dstt = jnp.stack([da, db])

    z_deg = jnp.zeros((NPAD, DEG_W), f32)
    ones_r = jnp.full((CHUNK, DEG_W), 1.0 / DEG_W, f32)
    z_agg = jnp.zeros((NPAD, H), jnp.bfloat16)

    deg_sc, agg_sc = _sc_kernels()
    degp = deg_sc(dstt, z_deg, ones_r)
    ub, dinv = _tc_a(xs, jnp.stack([W1a, W1b]), degp)

    b1 = jnp.stack([b1a, b1b])[:, None, :]
    bc1s = jnp.stack([bc1, bc1])[:, None, :]
    bc2s = jnp.stack([bc2, bc2])[:, None, :]

    agg1 = agg_sc(ub, srct, dstt, z_agg)
    ub2 = _tc_b(agg1, ub, dinv, b1, Wc1)
    agg2 = agg_sc(ub2, srct, dstt, z_agg)
    ub3 = _tc_b(agg2, ub2, dinv, bc1s, Wc2)
    agg3 = agg_sc(ub3, srct, dstt, z_agg)

    pooled, y = _tc_c(agg3, ub3, dinv, bc2s,
                      jnp.stack([batch, batch2]).astype(jnp.int32)[..., None],
                      jnp.stack([L1a, L1b]),
                      jnp.stack([bl1a, bl1b])[:, None, :],
                      jnp.stack([L2a, L2b]),
                      jnp.stack([bl2a, bl2b])[:, None, :])
    return (pooled, y)


# speedup vs baseline: 1.7083x; 1.0426x over previous
import functools

import jax
import jax.numpy as jnp
from jax import lax
from jax.experimental import pallas as pl
from jax.experimental.pallas import tpu as pltpu
from jax.experimental.pallas import tpu_sc as plsc

N = 10000
E = 320000
D = 128
H = 64
C = 10
G = 128

NC = 2
NS = 16
CHUNK = 128
EPT = E // NS
NBUF = 8
PREF = 4
NCHUNKS = NBUF * (-(-EPT // (CHUNK * NBUF)))
NG = NCHUNKS // NBUF
EPT_PAD = NCHUNKS * CHUNK
RPT = 632
NPAD = NS * RPT
DEG_W = 8
BN = 2000
NB = N // BN

@functools.cache
def _sc_kernels():
    mesh = plsc.VectorSubcoreMesh(
        core_axis_name="c", subcore_axis_name="s",
        num_cores=NC, num_subcores=NS)

    @functools.partial(
        pl.kernel,
        out_type=jax.ShapeDtypeStruct((NC, NPAD, DEG_W), jnp.float32),
        mesh=mesh,
        scratch_types=[
            pltpu.VMEM((NCHUNKS, CHUNK), jnp.int32),
            pltpu.VMEM((CHUNK, DEG_W), jnp.float32),
            pltpu.VMEM_SHARED((NPAD, DEG_W), jnp.float32),
            pltpu.SemaphoreType.DMA,
        ],
        compiler_params=pltpu.CompilerParams(use_tc_tiling_on_sc=False),
    )
    def deg_sc(dst_hbm, zeros_hbm, ones_hbm, out_hbm, idx_d, onesb, acc, sem):
        c = lax.axis_index("c")
        s = lax.axis_index("s")
        pltpu.sync_copy(dst_hbm.at[c, s], idx_d)
        pltpu.sync_copy(ones_hbm, onesb)
        pltpu.sync_copy(zeros_hbm.at[pl.ds(s * RPT, RPT)],
                        acc.at[pl.ds(s * RPT, RPT)])
        plsc.subcore_barrier()

        def fire(i):
            pltpu.async_copy(onesb, acc.at[idx_d.at[i]], sem, add=True)

        def wait_one():
            pltpu.make_async_copy(onesb, acc.at[idx_d.at[0]], sem).wait()

        for i in range(NBUF):
            fire(i)

        def body(i, carry):
            wait_one()
            fire(i)
            return carry

        lax.fori_loop(NBUF, NCHUNKS, body, 0)
        for _ in range(NBUF):
            wait_one()
        plsc.subcore_barrier()
        pltpu.sync_copy(acc.at[pl.ds(s * RPT, RPT)],
                        out_hbm.at[c, pl.ds(s * RPT, RPT)])

    @functools.partial(
        pl.kernel,
        out_type=jax.ShapeDtypeStruct((NC, NPAD, H), jnp.bfloat16),
        mesh=mesh,
        scratch_types=[
            pltpu.VMEM((NCHUNKS, CHUNK), jnp.int32),
            pltpu.VMEM((NCHUNKS, CHUNK), jnp.int32),
            [pltpu.VMEM((CHUNK, H), jnp.bfloat16) for _ in range(NBUF)],
            pltpu.VMEM_SHARED((NPAD, H), jnp.bfloat16),
            pltpu.SemaphoreType.DMA((NBUF,)),
            pltpu.SemaphoreType.DMA((NBUF,)),
        ],
        compiler_params=pltpu.CompilerParams(use_tc_tiling_on_sc=False),
    )
    def agg_sc(u_hbm, src_hbm, dst_hbm, zeros_hbm, out_hbm,
               idx_s, idx_d, rows, acc, gsem, ssem):
        c = lax.axis_index("c")
        s = lax.axis_index("s")
        pltpu.sync_copy(src_hbm.at[c, s], idx_s)
        pltpu.sync_copy(dst_hbm.at[c, s], idx_d)
        pltpu.sync_copy(zeros_hbm.at[pl.ds(s * RPT, RPT)],
                        acc.at[pl.ds(s * RPT, RPT)])
        plsc.subcore_barrier()

        def fire_g(i, j):
            pltpu.async_copy(u_hbm.at[idx_s.at[i]], rows[j], gsem.at[j])

        def wait_g(i, j):
            pltpu.make_async_copy(u_hbm.at[idx_s.at[i]], rows[j],
                                  gsem.at[j]).wait()

        def fire_s(i, j):
            pltpu.async_copy(rows[j], acc.at[idx_d.at[i]], ssem.at[j],
                             add=True)

        def wait_s(i, j):
            pltpu.make_async_copy(rows[j], acc.at[idx_d.at[i]],
                                  ssem.at[j]).wait()

        def step(it, j, first, last):
            i = it * NBUF + j
            k = i + PREF
            jk = (j + PREF) % NBUF
            if not last:
                if not (first and j < PREF):
                    wait_s(k - NBUF, jk)
                fire_g(k, jk)
            elif j < PREF:
                wait_s(k - NBUF, jk)
                fire_g(k, jk)
            wait_g(i, j)
            fire_s(i, j)

        for j in range(PREF):
            fire_g(j, j)
        for j in range(NBUF):
            step(0, j, True, False)

        def group(it, carry):
            for j in range(NBUF):
                step(it, j, False, False)
            return carry

        lax.fori_loop(1, NG - 1, group, 0)
        for j in range(NBUF):
            step(NG - 1, j, False, True)
        for j in range(NBUF):
            wait_s(NCHUNKS - NBUF + j, j)

        plsc.subcore_barrier()
        pltpu.sync_copy(acc.at[pl.ds(s * RPT, RPT)],
                        out_hbm.at[c, pl.ds(s * RPT, RPT)])

    return deg_sc, agg_sc


BP = BN // 2


def _tc_a(xp, x2p, Wbd, degpp):
    def body(x_ref, x2_ref, w_ref, degp_ref, ub_ref, dinv_ref):
        b = pl.program_id(0)
        deg_l = jnp.sum(degp_ref[0][:, :DEG_W], axis=1, keepdims=True) + 1.0
        deg_r = jnp.sum(degp_ref[0][:, DEG_W:], axis=1, keepdims=True) + 1.0
        dinvp = jnp.concatenate(
            [jnp.broadcast_to(lax.rsqrt(deg_l), (BP, H)),
             jnp.broadcast_to(lax.rsqrt(deg_r), (BP, H))], axis=1)
        xsel = jnp.where(b == 0, x_ref[...], x2_ref[...])
        xw = jnp.dot(xsel, w_ref[0], preferred_element_type=jnp.float32)
        ub_ref[...] = (xw * dinvp).astype(jnp.bfloat16)
        dinv_ref[...] = dinvp.astype(jnp.bfloat16)

    return pl.pallas_call(
        body,
        grid=(2, NB),
        in_specs=[
            pl.BlockSpec((BP, 2 * D), lambda b, i: (i, 0)),
            pl.BlockSpec((BP, 2 * D), lambda b, i: (i, 0)),
            pl.BlockSpec((1, 2 * D, 2 * H), lambda b, i: (b, 0, 0)),
            pl.BlockSpec((1, BP, 2 * DEG_W), lambda b, i: (b, i, 0)),
        ],
        out_specs=[
            pl.BlockSpec((BP, 2 * H), lambda b, i: (b * NB + i, 0)),
            pl.BlockSpec((BP, 2 * H), lambda b, i: (b * NB + i, 0)),
        ],
        out_shape=[
            jax.ShapeDtypeStruct((N, 2 * H), jnp.bfloat16),
            jax.ShapeDtypeStruct((N, 2 * H), jnp.bfloat16),
        ],
    )(xp, x2p, Wbd, degpp)


def _tc_b(aggp, ubp, dinvp, biasp, Wbd):
    def body(agg_ref, ub_ref, dinv_ref, b_ref, w_ref, un_ref):
        dinv = dinv_ref[...].astype(jnp.float32)
        pre = (agg_ref[0] + ub_ref[...]).astype(jnp.float32)
        h = jnp.maximum(dinv * pre + b_ref[0], 0.0)
        un = jnp.dot(h, w_ref[...],
                     preferred_element_type=jnp.float32) * dinv
        un_ref[...] = un.astype(jnp.bfloat16)

    return pl.pallas_call(
        body,
        grid=(2, NB),
        in_specs=[
            pl.BlockSpec((1, BP, 2 * H), lambda b, i: (b, i, 0)),
            pl.BlockSpec((BP, 2 * H), lambda b, i: (b * NB + i, 0)),
            pl.BlockSpec((BP, 2 * H), lambda b, i: (b * NB + i, 0)),
            pl.BlockSpec((1, 1, 2 * H), lambda b, i: (b, 0, 0)),
            pl.BlockSpec((2 * H, 2 * H), lambda b, i: (0, 0)),
        ],
        out_specs=pl.BlockSpec((BP, 2 * H), lambda b, i: (b * NB + i, 0)),
        out_shape=jax.ShapeDtypeStruct((N, 2 * H), jnp.bfloat16),
    )(aggp, ubp, dinvp, biasp, Wbd)


def _tc_c(aggp, ubp, dinvp, biasp, bts_e, bts_o, L1s, bl1s, L2s, bl2s):
    def body(agg_ref, ub_ref, dinv_ref, b_ref, bte_ref, bto_ref,
             l1_ref, bl1_ref, l2_ref, bl2_ref,
             pooled_ref, y_ref, s_acc, cnt_acc):
        i = pl.program_id(1)
        dinv = dinv_ref[...].astype(jnp.float32)
        pre = (agg_ref[0] + ub_ref[...]).astype(jnp.float32)
        h = jnp.maximum(dinv * pre + b_ref[0], 0.0)
        oh_e = (bte_ref[0] == lax.broadcasted_iota(jnp.int32, (BP, G), 1)
                ).astype(jnp.float32)
        oh_o = (bto_ref[0] == lax.broadcasted_iota(jnp.int32, (BP, G), 1)
                ).astype(jnp.float32)
        sp = (lax.dot_general(oh_e, h[:, :H], (((0,), (0,)), ((), ())),
                              preferred_element_type=jnp.float32)
              + lax.dot_general(oh_o, h[:, H:], (((0,), (0,)), ((), ())),
                                preferred_element_type=jnp.float32))
        cp = lax.dot_general(oh_e + oh_o, jnp.ones((BP, 1), jnp.float32),
                             (((0,), (0,)), ((), ())),
                             preferred_element_type=jnp.float32)

        @pl.when(i == 0)
        def _():
            s_acc[...] = sp
            cnt_acc[...] = cp

        @pl.when(i > 0)
        def _():
            s_acc[...] += sp
            cnt_acc[...] += cp

        @pl.when(i == NB - 1)
        def _():
            pooled = s_acc[...] / jnp.maximum(cnt_acc[...], 1.0)
            t = jnp.maximum(
                jnp.dot(pooled, l1_ref[0],
                        preferred_element_type=jnp.float32) + bl1_ref[0], 0.0)
            y_ref[0] = jnp.dot(t, l2_ref[0],
                               preferred_element_type=jnp.float32) + bl2_ref[0]
            pooled_ref[0] = pooled

    return pl.pallas_call(
        body,
        grid=(2, NB),
        in_specs=[
            pl.BlockSpec((1, BP, 2 * H), lambda b, i: (b, i, 0)),
            pl.BlockSpec((BP, 2 * H), lambda b, i: (b * NB + i, 0)),
            pl.BlockSpec((BP, 2 * H), lambda b, i: (b * NB + i, 0)),
            pl.BlockSpec((1, 1, 2 * H), lambda b, i: (b, 0, 0)),
            pl.BlockSpec((1, BP, 1), lambda b, i: (b, i, 0)),
            pl.BlockSpec((1, BP, 1), lambda b, i: (b, i, 0)),
            pl.BlockSpec((1, H, H), lambda b, i: (b, 0, 0)),
            pl.BlockSpec((1, 1, H), lambda b, i: (b, 0, 0)),
            pl.BlockSpec((1, H, C), lambda b, i: (b, 0, 0)),
            pl.BlockSpec((1, 1, C), lambda b, i: (b, 0, 0)),
        ],
        out_specs=[
            pl.BlockSpec((1, G, H), lambda b, i: (b, 0, 0)),
            pl.BlockSpec((1, G, C), lambda b, i: (b, 0, 0)),
        ],
        out_shape=[
            jax.ShapeDtypeStruct((2, G, H), jnp.float32),
            jax.ShapeDtypeStruct((2, G, C), jnp.float32),
        ],
        scratch_shapes=[
            pltpu.VMEM((G, H), jnp.float32),
            pltpu.VMEM((G, 1), jnp.float32),
        ],
    )(aggp, ubp, dinvp, biasp, bts_e, bts_o, L1s, bl1s, L2s, bl2s)


def _prep_edges(ei, soff):
    srcp = jnp.pad(ei[0].reshape(NS, EPT), ((0, 0), (0, EPT_PAD - EPT)),
                   constant_values=0) + soff
    dstp = jnp.pad(ei[1].reshape(NS, EPT), ((0, 0), (0, EPT_PAD - EPT)),
                   constant_values=N)
    return (srcp.reshape(NS, NCHUNKS, CHUNK).astype(jnp.int32),
            dstp.reshape(NS, NCHUNKS, CHUNK).astype(jnp.int32))


def _blockdiag(W):
    z = jnp.zeros_like(W)
    return jnp.concatenate(
        [jnp.concatenate([W, z], axis=1), jnp.concatenate([z, W], axis=1)],
        axis=0)


def kernel(x, edge_index, batch, x2, edge_index2, batch2,
           W1a, b1a, W1b, b1b, Wc1, bc1, Wc2, bc2,
           L1a, bl1a, L2a, bl2a, L1b, bl1b, L2b, bl2b):
    f32 = jnp.float32

    sa, da = _prep_edges(edge_index, 0)
    sb, db = _prep_edges(edge_index2, N)
    srct = jnp.stack([sa, sb])
    dstt = jnp.stack([da, db])

    z_deg = jnp.zeros((NPAD, DEG_W), f32)
    ones_r = jnp.full((CHUNK, DEG_W), 1.0 / DEG_W, f32)
    z_agg = jnp.zeros((NPAD, H), jnp.bfloat16)

    deg_sc, agg_sc = _sc_kernels()
    degp = deg_sc(dstt, z_deg, ones_r)

    ubp, dinvp = _tc_a(x.reshape(N // 2, 2 * D), x2.reshape(N // 2, 2 * D),
                       jnp.stack([_blockdiag(W1a), _blockdiag(W1b)]),
                       degp.reshape(2, NPAD // 2, 2 * DEG_W))

    b1p = jnp.stack([jnp.concatenate([b1a, b1a]),
                     jnp.concatenate([b1b, b1b])])[:, None, :]
    bc1p = jnp.tile(jnp.concatenate([bc1, bc1])[None, None, :], (2, 1, 1))
    bc2p = jnp.tile(jnp.concatenate([bc2, bc2])[None, None, :], (2, 1, 1))

    agg1 = agg_sc(ubp.reshape(2 * N, H), srct, dstt, z_agg)
    ub2p = _tc_b(agg1.reshape(2, NPAD // 2, 2 * H), ubp, dinvp, b1p,
                 _blockdiag(Wc1))
    agg2 = agg_sc(ub2p.reshape(2 * N, H), srct, dstt, z_agg)
    ub3p = _tc_b(agg2.reshape(2, NPAD // 2, 2 * H), ub2p, dinvp, bc1p,
                 _blockdiag(Wc2))
    agg3 = agg_sc(ub3p.reshape(2 * N, H), srct, dstt, z_agg)

    bts = jnp.stack([batch, batch2]).astype(jnp.int32)
    pooled, y = _tc_c(agg3.reshape(2, NPAD // 2, 2 * H), ub3p, dinvp, bc2p,
                      bts[:, 0::2, None], bts[:, 1::2, None],
                      jnp.stack([L1a, L1b]),
                      jnp.stack([bl1a, bl1b])[:, None, :],
                      jnp.stack([L2a, L2b]),
                      jnp.stack([bl2a, bl2b])[:, None, :])
    return (pooled, y)
